# baseline (speedup 1.0000x reference)
import functools
import math

import jax
import jax.numpy as jnp
import numpy as np
from jax import lax
from jax.experimental import pallas as pl
from jax.experimental.pallas import tpu as pltpu
from jax.experimental.pallas import tpu_sc as plsc


def _make_pe_np(max_length: int, d: int) -> np.ndarray:
    pe = np.zeros((max_length, d), dtype=np.float32)
    position = np.arange(0.0, max_length, dtype=np.float32)[:, None]
    div_term = np.exp(np.arange(0.0, d, 2, dtype=np.float32) * -(math.log(10000.0) / d))
    pe[:, 0::2] = np.sin(position * div_term)
    pe[:, 1::2] = np.cos(position * div_term)
    return pe


@functools.lru_cache(maxsize=None)
def _sc_gather_pack_fn(n: int, v: int, d: int, chunk: int):
    info = plsc.get_sparse_core_info()
    nc, ns = info.num_cores, info.num_subcores
    nw = nc * ns
    npairs = n // 2
    assert npairs % (nw * 2 * chunk) == 0 and chunk % 8 == 0
    p_per_w = npairs // nw
    n_iters = p_per_w // (2 * chunk)
    ngrp = d // info.num_lanes

    mesh = plsc.VectorSubcoreMesh(core_axis_name="c", subcore_axis_name="s")

    @functools.partial(
        pl.kernel,
        mesh=mesh,
        out_type=jax.ShapeDtypeStruct((npairs, d), jnp.int32),
        scratch_types=[
            pltpu.VMEM((2 * p_per_w,), jnp.int32),
            pltpu.VMEM((chunk, d), jnp.float32),
            pltpu.VMEM((chunk, d), jnp.float32),
            pltpu.VMEM((chunk, d), jnp.float32),
            pltpu.VMEM((chunk, d), jnp.float32),
            pltpu.VMEM((chunk, d), jnp.int32),
            pltpu.VMEM((chunk, d), jnp.int32),
            pltpu.SemaphoreType.DMA,
            pltpu.SemaphoreType.DMA,
            pltpu.SemaphoreType.DMA,
            pltpu.SemaphoreType.DMA,
        ],
        compiler_params=pltpu.CompilerParams(needs_layout_passes=False),
    )
    def sc_gather(table_hbm, idx_hbm, out_hbm, idx_v, ra0, ra1, rb0, rb1,
                  pk0, pk1, gsem, gsem2, w0, w1):
        wid = lax.axis_index("s") * nc + lax.axis_index("c")
        abase = wid * p_per_w
        bbase = npairs + wid * p_per_w
        pltpu.sync_copy(idx_hbm.at[pl.ds(abase, p_per_w)],
                        idx_v.at[pl.ds(0, p_per_w)])
        pltpu.sync_copy(idx_hbm.at[pl.ds(bbase, p_per_w)],
                        idx_v.at[pl.ds(p_per_w, p_per_w)])
        ras = (ra0, ra1)
        rbs = (rb0, rb1)
        pks = (pk0, pk1)
        wsem = (w0, w1)
        gsem_ = (gsem, gsem2)
        n_chunks = 2 * n_iters

        def issue_gather(i, s):
            pltpu.async_copy(
                table_hbm.at[idx_v.at[pl.ds(i * chunk, chunk)]],
                ras[s], gsem_[s])
            pltpu.async_copy(
                table_hbm.at[idx_v.at[pl.ds(p_per_w + i * chunk, chunk)]],
                rbs[s], gsem_[s])

        def wait_gather(s):
            pltpu.make_async_copy(table_hbm.at[pl.ds(0, chunk)],
                                  ras[s], gsem_[s]).wait()
            pltpu.make_async_copy(table_hbm.at[pl.ds(0, chunk)],
                                  rbs[s], gsem_[s]).wait()

        issue_gather(0, 0)

        def step(j, carry):
            for s in range(2):
                i = 2 * j + s

                @pl.when(i + 1 < n_chunks)
                def _():
                    issue_gather(i + 1, (s + 1) % 2)

                wait_gather(s)
                @pl.when(j > 0)
                def _():
                    prev = pl.multiple_of(abase + (i - 2) * chunk, 8)
                    pltpu.make_async_copy(
                        pks[s], out_hbm.at[pl.ds(prev, chunk)], wsem[s]
                    ).wait()

                @plsc.parallel_loop(0, chunk, unroll=4)
                def pack_pair(p):
                    for g in range(ngrp):
                        a = ras[s][p, pl.ds(16 * g, 16)]
                        bvec = rbs[s][p, pl.ds(16 * g, 16)]
                        w = plsc.bitcast(
                            plsc.pack(a, bvec,
                                      format=plsc.PackFormat.INTERLEAVED),
                            jnp.int32)
                        pks[s][p, pl.ds(16 * g, 16)] = w

                off = pl.multiple_of(abase + i * chunk, 8)
                pltpu.async_copy(pks[s], out_hbm.at[pl.ds(off, chunk)], wsem[s])
            return carry

        lax.fori_loop(0, n_iters, step, 0)
        for s in range(2):
            i = 2 * (n_iters - 1) + s
            off = pl.multiple_of(abase + i * chunk, 8)
            pltpu.make_async_copy(
                pks[s], out_hbm.at[pl.ds(off, chunk)], wsem[s]
            ).wait()

    return sc_gather


@functools.lru_cache(maxsize=None)
def _tc_dense2_fn(bh: int, l: int, d: int, bblk: int):
    assert bh % bblk == 0
    scale = math.sqrt(float(d))

    def body(pk_ref, pe_ref, wt_ref, bias_ref, out_ref):
        pk = pk_ref[...]
        pe = pe_ref[...][None]
        wt = wt_ref[...]
        rb = bias_ref[...] + 1.0
        halves = (
            lax.bitcast_convert_type(pk << 16, jnp.float32),
            lax.bitcast_convert_type(pk & jnp.int32(-65536), jnp.float32),
        )
        for half in range(2):
            x = halves[half] * scale + pe
            xf = x.reshape(bblk * l, d)
            r = jnp.dot(xf, wt, preferred_element_type=jnp.float32) + rb
            out_ref[half] = (xf * r).reshape(bblk, l, d)

    return pl.pallas_call(
        body,
        grid=(bh // bblk,),
        in_specs=[
            pl.BlockSpec((bblk, l, d), lambda i: (i, 0, 0)),
            pl.BlockSpec((l, d), lambda i: (0, 0)),
            pl.BlockSpec((d, d), lambda i: (0, 0)),
            pl.BlockSpec((1, d), lambda i: (0, 0)),
        ],
        out_specs=pl.BlockSpec((2, bblk, l, d), lambda i: (0, i, 0, 0)),
        out_shape=jax.ShapeDtypeStruct((2, bh, l, d), jnp.float32),
    )


def kernel(src, tok_embedding, linear_w, linear_b):
    b, l = src.shape
    v, d = tok_embedding.shape
    pe = jnp.asarray(_make_pe_np(512, d)[:l])
    wt = linear_w.T
    bias = linear_b.reshape(1, d)
    n = b * l
    pk = _sc_gather_pack_fn(n, v, d, chunk=80)(tok_embedding, src.reshape(-1))
    bh = b // 2
    z2 = _tc_dense2_fn(bh, l, d, bblk=32)(pk.reshape(bh, l, d), pe, wt, bias)
    return z2.reshape(b, l, d)


# speedup vs baseline: 1.0206x; 1.0206x over previous
import functools
import math

import jax
import jax.numpy as jnp
import numpy as np
from jax import lax
from jax.experimental import pallas as pl
from jax.experimental.pallas import tpu as pltpu
from jax.experimental.pallas import tpu_sc as plsc


def _make_pe_np(max_length: int, d: int) -> np.ndarray:
    pe = np.zeros((max_length, d), dtype=np.float32)
    position = np.arange(0.0, max_length, dtype=np.float32)[:, None]
    div_term = np.exp(np.arange(0.0, d, 2, dtype=np.float32) * -(math.log(10000.0) / d))
    pe[:, 0::2] = np.sin(position * div_term)
    pe[:, 1::2] = np.cos(position * div_term)
    return pe


@functools.lru_cache(maxsize=None)
def _sc_gather_pack_fn(n: int, v: int, d: int, chunk: int):
    info = plsc.get_sparse_core_info()
    nc, ns = info.num_cores, info.num_subcores
    nw = nc * ns
    npairs = n // 2
    assert npairs % (nw * 2 * chunk) == 0 and chunk % 8 == 0
    p_per_w = npairs // nw
    n_iters = p_per_w // (2 * chunk)
    ngrp = d // info.num_lanes

    mesh = plsc.VectorSubcoreMesh(core_axis_name="c", subcore_axis_name="s")

    @functools.partial(
        pl.kernel,
        mesh=mesh,
        out_type=jax.ShapeDtypeStruct((npairs, d), jnp.int32),
        scratch_types=[
            pltpu.VMEM((2 * p_per_w,), jnp.int32),
            pltpu.VMEM((chunk, d), jnp.float32),
            pltpu.VMEM((chunk, d), jnp.float32),
            pltpu.VMEM((chunk, d), jnp.float32),
            pltpu.VMEM((chunk, d), jnp.float32),
            pltpu.VMEM((chunk, d), jnp.int32),
            pltpu.VMEM((chunk, d), jnp.int32),
            pltpu.SemaphoreType.DMA,
            pltpu.SemaphoreType.DMA,
            pltpu.SemaphoreType.DMA,
            pltpu.SemaphoreType.DMA,
        ],
        compiler_params=pltpu.CompilerParams(needs_layout_passes=False),
    )
    def sc_gather(table_hbm, idx_hbm, out_hbm, idx_v, ra0, ra1, rb0, rb1,
                  pk0, pk1, gsem, gsem2, w0, w1):
        wid = lax.axis_index("s") * nc + lax.axis_index("c")
        abase = wid * p_per_w
        bbase = npairs + wid * p_per_w
        pltpu.sync_copy(idx_hbm.at[pl.ds(abase, p_per_w)],
                        idx_v.at[pl.ds(0, p_per_w)])
        pltpu.sync_copy(idx_hbm.at[pl.ds(bbase, p_per_w)],
                        idx_v.at[pl.ds(p_per_w, p_per_w)])
        ras = (ra0, ra1)
        rbs = (rb0, rb1)
        pks = (pk0, pk1)
        wsem = (w0, w1)
        gsem_ = (gsem, gsem2)
        n_chunks = 2 * n_iters

        def issue_gather(i, s):
            pltpu.async_copy(
                table_hbm.at[idx_v.at[pl.ds(i * chunk, chunk)]],
                ras[s], gsem_[s])
            pltpu.async_copy(
                table_hbm.at[idx_v.at[pl.ds(p_per_w + i * chunk, chunk)]],
                rbs[s], gsem_[s])

        def wait_gather(s):
            pltpu.make_async_copy(table_hbm.at[pl.ds(0, chunk)],
                                  ras[s], gsem_[s]).wait()
            pltpu.make_async_copy(table_hbm.at[pl.ds(0, chunk)],
                                  rbs[s], gsem_[s]).wait()

        issue_gather(0, 0)

        def step(j, carry):
            for s in range(2):
                i = 2 * j + s

                @pl.when(i + 1 < n_chunks)
                def _():
                    issue_gather(i + 1, (s + 1) % 2)

                wait_gather(s)
                @pl.when(j > 0)
                def _():
                    prev = pl.multiple_of(abase + (i - 2) * chunk, 8)
                    pltpu.make_async_copy(
                        pks[s], out_hbm.at[pl.ds(prev, chunk)], wsem[s]
                    ).wait()

                @plsc.parallel_loop(0, chunk, unroll=4)
                def pack_pair(p):
                    for g in range(ngrp):
                        a = ras[s][p, pl.ds(16 * g, 16)]
                        bvec = rbs[s][p, pl.ds(16 * g, 16)]
                        w = plsc.bitcast(
                            plsc.pack(a, bvec,
                                      format=plsc.PackFormat.INTERLEAVED),
                            jnp.int32)
                        pks[s][p, pl.ds(16 * g, 16)] = w

                off = pl.multiple_of(abase + i * chunk, 8)
                pltpu.async_copy(pks[s], out_hbm.at[pl.ds(off, chunk)], wsem[s])
            return carry

        lax.fori_loop(0, n_iters, step, 0)
        for s in range(2):
            i = 2 * (n_iters - 1) + s
            off = pl.multiple_of(abase + i * chunk, 8)
            pltpu.make_async_copy(
                pks[s], out_hbm.at[pl.ds(off, chunk)], wsem[s]
            ).wait()

    return sc_gather


@functools.lru_cache(maxsize=None)
def _tc_dense2_fn(bh: int, l: int, d: int, bblk: int):
    assert bh % bblk == 0
    scale = math.sqrt(float(d))

    def body(pk_ref, pe_ref, wt_ref, bias_ref, out_ref):
        pk = pk_ref[...]
        pe = pe_ref[...][None]
        wt = wt_ref[...]
        rb = bias_ref[...] + 1.0
        halves = (
            lax.bitcast_convert_type(pk << 16, jnp.float32),
            lax.bitcast_convert_type(pk & jnp.int32(-65536), jnp.float32),
        )
        for half in range(2):
            x = halves[half] * scale + pe
            xf = x.reshape(bblk * l, d)
            r = jnp.dot(xf, wt, preferred_element_type=jnp.float32) + rb
            out_ref[half] = (xf * r).reshape(bblk, l, d)

    return pl.pallas_call(
        body,
        grid=(bh // bblk,),
        in_specs=[
            pl.BlockSpec((bblk, l, d), lambda i: (i, 0, 0)),
            pl.BlockSpec((l, d), lambda i: (0, 0)),
            pl.BlockSpec((d, d), lambda i: (0, 0)),
            pl.BlockSpec((1, d), lambda i: (0, 0)),
        ],
        out_specs=pl.BlockSpec((2, bblk, l, d), lambda i: (0, i, 0, 0)),
        out_shape=jax.ShapeDtypeStruct((2, bh, l, d), jnp.float32),
    )


def kernel(src, tok_embedding, linear_w, linear_b):
    b, l = src.shape
    v, d = tok_embedding.shape
    pe = jnp.asarray(_make_pe_np(512, d)[:l])
    wt = linear_w.T
    bias = linear_b.reshape(1, d)
    n = b * l
    pk = _sc_gather_pack_fn(n, v, d, chunk=160)(tok_embedding, src.reshape(-1))
    bh = b // 2
    z2 = _tc_dense2_fn(bh, l, d, bblk=32)(pk.reshape(bh, l, d), pe, wt, bias)
    return z2.reshape(b, l, d)


# speedup vs baseline: 1.0447x; 1.0236x over previous
import functools
import math

import jax
import jax.numpy as jnp
import numpy as np
from jax import lax
from jax.experimental import pallas as pl
from jax.experimental.pallas import tpu as pltpu
from jax.experimental.pallas import tpu_sc as plsc


def _make_pe_np(max_length: int, d: int) -> np.ndarray:
    pe = np.zeros((max_length, d), dtype=np.float32)
    position = np.arange(0.0, max_length, dtype=np.float32)[:, None]
    div_term = np.exp(np.arange(0.0, d, 2, dtype=np.float32) * -(math.log(10000.0) / d))
    pe[:, 0::2] = np.sin(position * div_term)
    pe[:, 1::2] = np.cos(position * div_term)
    return pe


@functools.lru_cache(maxsize=None)
def _sc_gather_pack_fn(n: int, v: int, d: int, chunk: int):
    info = plsc.get_sparse_core_info()
    nc, ns = info.num_cores, info.num_subcores
    nw = nc * ns
    npairs = n // 2
    assert npairs % (nw * 2 * chunk) == 0 and chunk % 8 == 0
    p_per_w = npairs // nw
    n_iters = p_per_w // (2 * chunk)
    ngrp = d // info.num_lanes

    mesh = plsc.VectorSubcoreMesh(core_axis_name="c", subcore_axis_name="s")

    @functools.partial(
        pl.kernel,
        mesh=mesh,
        out_type=jax.ShapeDtypeStruct((npairs, d), jnp.int32),
        scratch_types=[
            pltpu.VMEM((2 * p_per_w,), jnp.int32),
            pltpu.VMEM((chunk, d), jnp.float32),
            pltpu.VMEM((chunk, d), jnp.float32),
            pltpu.VMEM((chunk, d), jnp.float32),
            pltpu.VMEM((chunk, d), jnp.float32),
            pltpu.VMEM((chunk, d), jnp.int32),
            pltpu.VMEM((chunk, d), jnp.int32),
            pltpu.SemaphoreType.DMA,
            pltpu.SemaphoreType.DMA,
            pltpu.SemaphoreType.DMA,
            pltpu.SemaphoreType.DMA,
        ],
        compiler_params=pltpu.CompilerParams(needs_layout_passes=False),
    )
    def sc_gather(table_hbm, idx_hbm, out_hbm, idx_v, ra0, ra1, rb0, rb1,
                  pk0, pk1, gsem, gsem2, w0, w1):
        wid = lax.axis_index("s") * nc + lax.axis_index("c")
        abase = wid * p_per_w
        bbase = npairs + wid * p_per_w
        pltpu.sync_copy(idx_hbm.at[pl.ds(abase, p_per_w)],
                        idx_v.at[pl.ds(0, p_per_w)])
        pltpu.sync_copy(idx_hbm.at[pl.ds(bbase, p_per_w)],
                        idx_v.at[pl.ds(p_per_w, p_per_w)])
        ras = (ra0, ra1)
        rbs = (rb0, rb1)
        pks = (pk0, pk1)
        wsem = (w0, w1)
        gsem_ = (gsem, gsem2)
        n_chunks = 2 * n_iters

        def issue_gather(i, s):
            pltpu.async_copy(
                table_hbm.at[idx_v.at[pl.ds(i * chunk, chunk)]],
                ras[s], gsem_[s])
            pltpu.async_copy(
                table_hbm.at[idx_v.at[pl.ds(p_per_w + i * chunk, chunk)]],
                rbs[s], gsem_[s])

        def wait_gather(s):
            pltpu.make_async_copy(table_hbm.at[pl.ds(0, chunk)],
                                  ras[s], gsem_[s]).wait()
            pltpu.make_async_copy(table_hbm.at[pl.ds(0, chunk)],
                                  rbs[s], gsem_[s]).wait()

        issue_gather(0, 0)

        def step(j, carry):
            for s in range(2):
                i = 2 * j + s

                @pl.when(i + 1 < n_chunks)
                def _():
                    issue_gather(i + 1, (s + 1) % 2)

                wait_gather(s)
                @pl.when(j > 0)
                def _():
                    prev = pl.multiple_of(abase + (i - 2) * chunk, 8)
                    pltpu.make_async_copy(
                        pks[s], out_hbm.at[pl.ds(prev, chunk)], wsem[s]
                    ).wait()

                @plsc.parallel_loop(0, chunk, unroll=4)
                def pack_pair(p):
                    for g in range(ngrp):
                        a = ras[s][p, pl.ds(16 * g, 16)]
                        bvec = rbs[s][p, pl.ds(16 * g, 16)]
                        w = plsc.bitcast(
                            plsc.pack(a, bvec,
                                      format=plsc.PackFormat.INTERLEAVED),
                            jnp.int32)
                        pks[s][p, pl.ds(16 * g, 16)] = w

                off = pl.multiple_of(abase + i * chunk, 8)
                pltpu.async_copy(pks[s], out_hbm.at[pl.ds(off, chunk)], wsem[s])
            return carry

        lax.fori_loop(0, n_iters, step, 0)
        for s in range(2):
            i = 2 * (n_iters - 1) + s
            off = pl.multiple_of(abase + i * chunk, 8)
            pltpu.make_async_copy(
                pks[s], out_hbm.at[pl.ds(off, chunk)], wsem[s]
            ).wait()

    return sc_gather


@functools.lru_cache(maxsize=None)
def _tc_dense2_fn(bh: int, l: int, d: int, bblk: int):
    assert bh % bblk == 0
    scale = math.sqrt(float(d))

    def body(pk_ref, pe_ref, wt_ref, bias_ref, out_ref):
        pk = pk_ref[...]
        pe = pe_ref[...][None]
        wt = wt_ref[...]
        rb = bias_ref[...] + 1.0
        halves = (
            lax.bitcast_convert_type(pk << 16, jnp.float32),
            lax.bitcast_convert_type(pk & jnp.int32(-65536), jnp.float32),
        )
        for half in range(2):
            x = halves[half] * scale + pe
            xf = x.reshape(bblk * l, d)
            r = jnp.dot(xf, wt, preferred_element_type=jnp.float32) + rb
            out_ref[half] = (xf * r).reshape(bblk, l, d)

    return pl.pallas_call(
        body,
        grid=(bh // bblk,),
        in_specs=[
            pl.BlockSpec((bblk, l, d), lambda i: (i, 0, 0)),
            pl.BlockSpec((l, d), lambda i: (0, 0)),
            pl.BlockSpec((d, d), lambda i: (0, 0)),
            pl.BlockSpec((1, d), lambda i: (0, 0)),
        ],
        out_specs=pl.BlockSpec((2, bblk, l, d), lambda i: (0, i, 0, 0)),
        out_shape=jax.ShapeDtypeStruct((2, bh, l, d), jnp.float32),
    )


def kernel(src, tok_embedding, linear_w, linear_b):
    b, l = src.shape
    v, d = tok_embedding.shape
    pe = jnp.asarray(_make_pe_np(512, d)[:l])
    wt = linear_w.T
    bias = linear_b.reshape(1, d)
    n = b * l
    pk = _sc_gather_pack_fn(n, v, d, chunk=160)(tok_embedding, src.reshape(-1))
    bh = b // 2
    z2 = _tc_dense2_fn(bh, l, d, bblk=64)(pk.reshape(bh, l, d), pe, wt, bias)
    return z2.reshape(b, l, d)
